# Initial kernel scaffold; baseline (speedup 1.0000x reference)
import functools

import jax
import jax.numpy as jnp
from jax import lax
from jax.experimental import pallas as pl
from jax.experimental.pallas import tpu as pltpu
from jax.experimental.pallas import tpu_sc as plsc

NUM_EMBEDDINGS = 1000000
EMBEDDING_DIM = 64

IDX_ROW = 128
CHUNK_ROWS = 8
NC = 2
NS = 16
NW = NC * NS


def _make_gather(B_flat):
    assert B_flat % (NW * IDX_ROW * CHUNK_ROWS) == 0
    rows_total = B_flat // IDX_ROW
    rows_per_w = rows_total // NW
    n_chunks = rows_per_w // CHUNK_ROWS
    chunk_elems = CHUNK_ROWS * IDX_ROW

    mesh = plsc.VectorSubcoreMesh(core_axis_name="c", subcore_axis_name="s")

    @functools.partial(
        pl.kernel,
        out_type=jax.ShapeDtypeStruct((B_flat, EMBEDDING_DIM), jnp.float32),
        mesh=mesh,
        scratch_types=[
            pltpu.VMEM((CHUNK_ROWS, IDX_ROW), jnp.int32),
            pltpu.VMEM((chunk_elems, EMBEDDING_DIM), jnp.float32),
            pltpu.SemaphoreType.DMA,
        ],
    )
    def gather_kernel(table_hbm, idx_hbm, out_hbm, idx_v, rows_v, sem):
        wid = lax.axis_index("s") * NC + lax.axis_index("c")
        row0 = wid * rows_per_w

        def chunk_body(g, _):
            r0 = row0 + g * CHUNK_ROWS
            pltpu.sync_copy(idx_hbm.at[pl.ds(r0, CHUNK_ROWS)], idx_v)
            copies = []
            for j in range(CHUNK_ROWS):
                copies.append(
                    pltpu.async_copy(
                        table_hbm.at[idx_v.at[j]],
                        rows_v.at[pl.ds(j * IDX_ROW, IDX_ROW)],
                        sem,
                    )
                )
            for c in copies:
                c.wait()
            pltpu.sync_copy(
                rows_v, out_hbm.at[pl.ds(r0 * IDX_ROW, chunk_elems)]
            )
            return ()

        lax.fori_loop(0, n_chunks, chunk_body, (), unroll=False)

    return gather_kernel


def kernel(token_ids, embed_weights):
    B, S = token_ids.shape
    B_flat = B * S
    idx2d = token_ids.reshape(B_flat // IDX_ROW, IDX_ROW).astype(jnp.int32)
    out = _make_gather(B_flat)(embed_weights, idx2d)
    return out.reshape(B, S, EMBEDDING_DIM)


# speedup vs baseline: 1.8450x; 1.8450x over previous
import functools

import jax
import jax.numpy as jnp
from jax import lax
from jax.experimental import pallas as pl
from jax.experimental.pallas import tpu as pltpu
from jax.experimental.pallas import tpu_sc as plsc

NUM_EMBEDDINGS = 1000000
EMBEDDING_DIM = 64

IDX_ROW = 128
CHUNK_ROWS = 8
NC = 2
NS = 16
NW = NC * NS


def _make_gather(B_flat):
    assert B_flat % (NW * IDX_ROW * CHUNK_ROWS) == 0
    rows_total = B_flat // IDX_ROW
    rows_per_w = rows_total // NW
    n_chunks = rows_per_w // CHUNK_ROWS
    chunk_elems = CHUNK_ROWS * IDX_ROW

    mesh = plsc.VectorSubcoreMesh(core_axis_name="c", subcore_axis_name="s")

    @functools.partial(
        pl.kernel,
        out_type=jax.ShapeDtypeStruct((B_flat, EMBEDDING_DIM), jnp.float32),
        mesh=mesh,
        scratch_types=[
            pltpu.VMEM((CHUNK_ROWS, IDX_ROW), jnp.int32),
            pltpu.VMEM((chunk_elems, EMBEDDING_DIM), jnp.float32),
            pltpu.SemaphoreType.DMA,
        ],
        compiler_params=pltpu.CompilerParams(use_tc_tiling_on_sc=False),
    )
    def gather_kernel(table_hbm, idx_hbm, out_hbm, idx_v, rows_v, sem):
        wid = lax.axis_index("s") * NC + lax.axis_index("c")
        row0 = wid * rows_per_w

        def chunk_body(g, _):
            r0 = row0 + g * CHUNK_ROWS
            pltpu.sync_copy(idx_hbm.at[pl.ds(r0, CHUNK_ROWS)], idx_v)
            copies = []
            for j in range(CHUNK_ROWS):
                copies.append(
                    pltpu.async_copy(
                        table_hbm.at[idx_v.at[j]],
                        rows_v.at[pl.ds(j * IDX_ROW, IDX_ROW)],
                        sem,
                    )
                )
            for c in copies:
                c.wait()
            pltpu.sync_copy(
                rows_v, out_hbm.at[pl.ds(r0 * IDX_ROW, chunk_elems)]
            )
            return ()

        lax.fori_loop(0, n_chunks, chunk_body, (), unroll=False)

    return gather_kernel


def kernel(token_ids, embed_weights):
    B, S = token_ids.shape
    B_flat = B * S
    idx2d = token_ids.reshape(B_flat // IDX_ROW, IDX_ROW).astype(jnp.int32)
    out = _make_gather(B_flat)(embed_weights, idx2d)
    return out.reshape(B, S, EMBEDDING_DIM)
